# Initial kernel scaffold; baseline (speedup 1.0000x reference)
import jax
import jax.numpy as jnp
from jax.experimental import pallas as pl


def kernel(a_norm, x, w, b):
    raise NotImplementedError("write your pallas kernel here")


# speedup vs baseline: 6.2359x; 6.2359x over previous
import jax
import jax.numpy as jnp
from jax.experimental import pallas as pl
from jax.experimental.pallas import tpu as pltpu


def _round_up(v, m):
    return (v + m - 1) // m * m


def _xw_kernel(x_ref, w_ref, o_ref):
    x_bf = x_ref[...].astype(jnp.bfloat16)
    w_bf = w_ref[...].astype(jnp.bfloat16)
    o_ref[...] = jnp.dot(
        x_bf, w_bf, preferred_element_type=jnp.float32
    ).astype(jnp.bfloat16)


def _xw(x, w, *, tm):
    n, h = x.shape
    return pl.pallas_call(
        _xw_kernel,
        out_shape=jax.ShapeDtypeStruct((n, h), jnp.bfloat16),
        grid=(n // tm,),
        in_specs=[
            pl.BlockSpec((tm, h), lambda i: (i, 0)),
            pl.BlockSpec((h, h), lambda i: (0, 0)),
        ],
        out_specs=pl.BlockSpec((tm, h), lambda i: (i, 0)),
        compiler_params=pltpu.CompilerParams(
            dimension_semantics=("parallel",),
            vmem_limit_bytes=100 * 1024 * 1024,
        ),
    )(x, w)


def _agg_kernel(a_ref, xw_ref, b_ref, o_ref):
    a_bf = a_ref[...].astype(jnp.bfloat16)
    acc = jnp.dot(a_bf, xw_ref[...], preferred_element_type=jnp.float32)
    o_ref[...] = jnp.maximum(acc + b_ref[...], 0.0)


def _aggregate(a, xw, b2d, *, tm):
    n, h = xw.shape
    return pl.pallas_call(
        _agg_kernel,
        out_shape=jax.ShapeDtypeStruct((n, h), jnp.float32),
        grid=(n // tm,),
        in_specs=[
            pl.BlockSpec((tm, n), lambda i: (i, 0)),
            pl.BlockSpec((n, h), lambda i: (0, 0)),
            pl.BlockSpec((1, h), lambda i: (0, 0)),
        ],
        out_specs=pl.BlockSpec((tm, h), lambda i: (i, 0)),
        compiler_params=pltpu.CompilerParams(
            dimension_semantics=("parallel",),
            vmem_limit_bytes=100 * 1024 * 1024,
        ),
    )(a, xw, b2d)


def kernel(a_norm, x, w, b):
    n, h = x.shape
    n_pad = _round_up(n, 256)
    h_pad = _round_up(h, 128)
    if n_pad != n or h_pad != h:
        a_norm = jnp.pad(a_norm, ((0, n_pad - n), (0, n_pad - n)))
        x = jnp.pad(x, ((0, n_pad - n), (0, h_pad - h)))
        w = jnp.pad(w, ((0, h_pad - h), (0, h_pad - h)))
        b = jnp.pad(b, (0, h_pad - h))
    b2d = b.reshape(1, h_pad).astype(jnp.float32)

    xw = _xw(x, w, tm=n_pad // 4)
    out = _aggregate(a_norm, xw, b2d, tm=256)
    return out[:n, :h]


# speedup vs baseline: 6.7956x; 1.0898x over previous
import functools

import jax
import jax.numpy as jnp
from jax.experimental import pallas as pl
from jax.experimental.pallas import tpu as pltpu


def _round_up(v, m):
    return (v + m - 1) // m * m


def _fused_kernel(x_ref, w_ref, b_ref, a_ref, o_ref, xw_ref):
    i = pl.program_id(0)

    @pl.when(i == 0)
    def _():
        xw_ref[...] = jnp.dot(
            x_ref[...].astype(jnp.bfloat16),
            w_ref[...].astype(jnp.bfloat16),
            preferred_element_type=jnp.float32,
        ).astype(jnp.bfloat16)

    @pl.when(i > 0)
    def _():
        a_bf = a_ref[...].astype(jnp.bfloat16)
        acc = jnp.dot(a_bf, xw_ref[...], preferred_element_type=jnp.float32)
        o_ref[...] = jnp.maximum(acc + b_ref[...], 0.0)


def _gcn_fused(a, x, w, b2d, *, tm):
    n, h = x.shape
    steps = n // tm

    def a_idx(i):
        return (jnp.maximum(i - 1, 0), 0)

    def o_idx(i):
        return (jnp.maximum(i - 1, 0), 0)

    return pl.pallas_call(
        _fused_kernel,
        out_shape=jax.ShapeDtypeStruct((n, h), jnp.float32),
        grid=(steps + 1,),
        in_specs=[
            pl.BlockSpec((n, h), lambda i: (0, 0)),
            pl.BlockSpec((h, h), lambda i: (0, 0)),
            pl.BlockSpec((1, h), lambda i: (0, 0)),
            pl.BlockSpec((tm, n), a_idx),
        ],
        out_specs=pl.BlockSpec((tm, h), o_idx),
        scratch_shapes=[pltpu.VMEM((n, h), jnp.bfloat16)],
        compiler_params=pltpu.CompilerParams(
            dimension_semantics=("arbitrary",),
            vmem_limit_bytes=100 * 1024 * 1024,
        ),
    )(x, w, b2d, a)


def kernel(a_norm, x, w, b):
    n, h = x.shape
    n_pad = _round_up(n, 256)
    h_pad = _round_up(h, 128)
    if n_pad != n or h_pad != h:
        a_norm = jnp.pad(a_norm, ((0, n_pad - n), (0, n_pad - n)))
        x = jnp.pad(x, ((0, n_pad - n), (0, h_pad - h)))
        w = jnp.pad(w, ((0, h_pad - h), (0, h_pad - h)))
        b = jnp.pad(b, (0, h_pad - h))
    b2d = b.reshape(1, h_pad).astype(jnp.float32)

    out = _gcn_fused(a_norm, x, w, b2d, tm=256)
    return out[:n, :h]


# speedup vs baseline: 7.5789x; 1.1153x over previous
import functools

import jax
import jax.numpy as jnp
from jax.experimental import pallas as pl
from jax.experimental.pallas import tpu as pltpu


def _round_up(v, m):
    return (v + m - 1) // m * m


def _fused_kernel(x_ref, w_ref, b_ref, a_ref, o_ref, xw_ref):
    i = pl.program_id(0)

    @pl.when(i == 0)
    def _():
        xw_ref[...] = jnp.dot(
            x_ref[...].astype(jnp.bfloat16),
            w_ref[...].astype(jnp.bfloat16),
            preferred_element_type=jnp.float32,
        ).astype(jnp.bfloat16)

    @pl.when(i > 0)
    def _():
        a_bf = a_ref[...].astype(jnp.bfloat16)
        acc = jnp.dot(a_bf, xw_ref[...], preferred_element_type=jnp.float32)
        o_ref[...] = jnp.maximum(acc + b_ref[...], 0.0)


def _gcn_fused(a, x, w, b2d, *, tm):
    n, h = x.shape
    steps = n // tm

    def a_idx(i):
        return (jnp.maximum(i - 1, 0), 0)

    def o_idx(i):
        return (jnp.maximum(i - 1, 0), 0)

    return pl.pallas_call(
        _fused_kernel,
        out_shape=jax.ShapeDtypeStruct((n, h), jnp.float32),
        grid=(steps + 1,),
        in_specs=[
            pl.BlockSpec((n, h), lambda i: (0, 0)),
            pl.BlockSpec((h, h), lambda i: (0, 0)),
            pl.BlockSpec((1, h), lambda i: (0, 0)),
            pl.BlockSpec((tm, n), a_idx),
        ],
        out_specs=pl.BlockSpec((tm, h), o_idx),
        scratch_shapes=[pltpu.VMEM((n, h), jnp.bfloat16)],
        compiler_params=pltpu.CompilerParams(
            dimension_semantics=("arbitrary",),
            vmem_limit_bytes=100 * 1024 * 1024,
        ),
    )(x, w, b2d, a)


def kernel(a_norm, x, w, b):
    n, h = x.shape
    n_pad = _round_up(n, 512)
    h_pad = _round_up(h, 128)
    if n_pad != n or h_pad != h:
        a_norm = jnp.pad(a_norm, ((0, n_pad - n), (0, n_pad - n)))
        x = jnp.pad(x, ((0, n_pad - n), (0, h_pad - h)))
        w = jnp.pad(w, ((0, h_pad - h), (0, h_pad - h)))
        b = jnp.pad(b, (0, h_pad - h))
    b2d = b.reshape(1, h_pad).astype(jnp.float32)

    out = _gcn_fused(a_norm, x, w, b2d, tm=512)
    return out[:n, :h]
